# baseline (device time: 80065 ns/iter reference)
import os

import jax
import jax.numpy as jnp
from jax import lax
from jax.experimental import pallas as pl
from jax.experimental.pallas import tpu as pltpu

NZ = 4
_MODE = os.environ.get("KMODE", "full")
_MESH = pl.DeviceIdType.MESH


def kernel(Q, K, V):
    b, s, h, d = Q.shape
    scale = d ** -0.5
    half = s // 2

    def body(q_ref, kv_ref, o_ref, kvbuf, l_ref,
             upsend0, upsend1, dnsend0, dnsend1,
             xsd0, xsd1, ysd0, ysd1, yfwd, xfwd,
             zrecv0, zrecv1, xrecvd0, xrecvd1, yrecvd0, yrecvd1,
             yrecvf, xrecvf):
        my_x = lax.axis_index("x")
        my_y = lax.axis_index("y")
        my_z = lax.axis_index("z")
        has_up = my_z < NZ - 1
        has_dn = my_z > 0
        up_dev = (my_x, my_y, jnp.minimum(my_z + 1, NZ - 1))
        dn_dev = (my_x, my_y, jnp.maximum(my_z - 1, 0))
        x_dev = (1 - my_x, my_y, my_z)
        y_dev = (my_x, 1 - my_y, my_z)
        self_dev = (my_x, my_y, my_z)

        bar = pltpu.get_barrier_semaphore()

        @pl.when(has_dn)
        def _():
            pl.semaphore_signal(bar, inc=1, device_id=dn_dev,
                                device_id_type=_MESH)

        @pl.when(has_up)
        def _():
            pl.semaphore_signal(bar, inc=1, device_id=up_dev,
                                device_id_type=_MESH)

        pl.semaphore_signal(bar, inc=1, device_id=x_dev, device_id_type=_MESH)
        pl.semaphore_signal(bar, inc=1, device_id=y_dev, device_id_type=_MESH)
        pl.semaphore_wait(bar, 2)

        @pl.when(has_dn)
        def _():
            pl.semaphore_wait(bar, 1)

        @pl.when(has_up)
        def _():
            pl.semaphore_wait(bar, 1)

        kvbuf[my_z] = kv_ref[...]

        sends = []

        def launch(cond, src, dev, ssem, rsem):
            c = pltpu.make_async_remote_copy(
                src_ref=src, dst_ref=src, send_sem=ssem, recv_sem=rsem,
                device_id=dev, device_id_type=_MESH)

            @pl.when(cond)
            def _():
                c.start()

            sends.append((cond, c))

        def wait_recv(cond, dst, rsem):
            c = pltpu.make_async_remote_copy(
                src_ref=dst, dst_ref=dst, send_sem=rsem, recv_sem=rsem,
                device_id=self_dev, device_id_type=_MESH)

            @pl.when(cond)
            def _():
                c.wait_recv()

        def remote_origins():
            for delta in range(1, NZ):
                yield (my_z - delta >= 0, jnp.maximum(my_z - delta, 0))
                yield (my_z + delta <= NZ - 1,
                       jnp.minimum(my_z + delta, NZ - 1))

        def attn_accum(get_k, get_v, init):
            for bi in range(b):
                for hi in range(h):
                    q = q_ref[bi, hi]
                    p = jnp.exp(lax.dot_general(
                        q, get_k(bi, hi), (((1,), (1,)), ((), ())),
                        preferred_element_type=jnp.float32) * scale)
                    lsum = jnp.sum(p, axis=1, keepdims=True)
                    pv = lax.dot_general(
                        p, get_v(bi, hi), (((1,), (0,)), ((), ())),
                        preferred_element_type=jnp.float32)
                    if init:
                        l_ref[bi, hi] = lsum
                        o_ref[bi, hi] = pv
                    else:
                        l_ref[bi, hi] += lsum
                        o_ref[bi, hi] += pv

        def finish_origin(cond, o):
            wait_recv(cond,
                      kvbuf.at[o, 1 - my_x, my_y, :, pl.ds(half, half)],
                      xrecvd1.at[o])
            wait_recv(cond,
                      kvbuf.at[o, my_x, 1 - my_y, :, pl.ds(0, half)],
                      yrecvd0.at[o])
            wait_recv(cond,
                      kvbuf.at[o, 1 - my_x, 1 - my_y, :, pl.ds(0, half)],
                      yrecvf.at[o])
            wait_recv(cond,
                      kvbuf.at[o, 1 - my_x, 1 - my_y, :, pl.ds(half, half)],
                      xrecvf.at[o])

            if _MODE != "comm_only":
                @pl.when(cond)
                def _():
                    attn_accum(lambda bi, hi: kvbuf[o, bi, 0, hi],
                               lambda bi, hi: kvbuf[o, bi, 1, hi],
                               init=False)

        def fwd_process(cond, o):
            wait_recv(cond, kvbuf.at[o, 1 - my_x, my_y, :, pl.ds(0, half)],
                      xrecvd0.at[o])
            launch(cond, kvbuf.at[o, 1 - my_x, my_y, :, pl.ds(0, half)],
                   y_dev, yfwd.at[o], yrecvf.at[o])
            wait_recv(cond,
                      kvbuf.at[o, my_x, 1 - my_y, :, pl.ds(half, half)],
                      yrecvd1.at[o])
            launch(cond,
                   kvbuf.at[o, my_x, 1 - my_y, :, pl.ds(half, half)],
                   x_dev, xfwd.at[o], xrecvf.at[o])

        do_compute = _MODE != "comm_only"

        if _MODE == "compute_only":
            attn_accum(lambda bi, hi: kv_ref[bi, 0, hi],
                       lambda bi, hi: kv_ref[bi, 1, hi], init=True)
        else:
            def p0(o):
                return kvbuf.at[o, my_x, my_y, :, pl.ds(0, half)]

            def p1(o):
                return kvbuf.at[o, my_x, my_y, :, pl.ds(half, half)]

            pend_fwd = []
            pend_fin = []
            for t in range(NZ - 1):
                uo = my_z - t
                uoc = jnp.maximum(uo, 0)
                do = my_z + t
                doc = jnp.minimum(do, NZ - 1)
                up_c = has_up & (uo >= 0)
                dn_c = has_dn & (do <= NZ - 1)
                rb = my_z - 1 - t
                rbc = jnp.maximum(rb, 0)
                ra = my_z + 1 + t
                rac = jnp.minimum(ra, NZ - 1)
                events = ((rb >= 0, rbc), (ra <= NZ - 1, rac))

                launch(up_c, p0(uoc), up_dev, upsend0.at[t], zrecv0.at[uoc])
                launch(dn_c, p0(doc), dn_dev, dnsend0.at[t], zrecv0.at[doc])
                if t == 0 and do_compute:
                    attn_accum(lambda bi, hi: kv_ref[bi, 0, hi],
                               lambda bi, hi: kv_ref[bi, 1, hi],
                               init=True)
                for cond, oc in events:
                    wait_recv(cond, p0(oc), zrecv0.at[oc])
                    launch(cond, p0(oc), x_dev, xsd0.at[oc], xrecvd0.at[oc])

                launch(up_c, p1(uoc), up_dev, upsend1.at[t], zrecv1.at[uoc])
                launch(dn_c, p1(doc), dn_dev, dnsend1.at[t], zrecv1.at[doc])
                fin_now = pend_fin
                pend_fin = []
                for cond, oc in pend_fwd:
                    fwd_process(cond, oc)
                    pend_fin.append((cond, oc))
                pend_fwd = []
                for cond, oc in fin_now:
                    finish_origin(cond, oc)
                for cond, oc in events:
                    wait_recv(cond, p1(oc), zrecv1.at[oc])
                    launch(cond, p1(oc), y_dev, ysd1.at[oc], yrecvd1.at[oc])
                    launch(cond, p1(oc), x_dev, xsd1.at[oc], xrecvd1.at[oc])
                    launch(cond, p0(oc), y_dev, ysd0.at[oc], yrecvd0.at[oc])
                    pend_fwd.append((cond, oc))

            for cond, oc in pend_fwd:
                fwd_process(cond, oc)
                pend_fin.append((cond, oc))
            for cond, oc in pend_fin:
                finish_origin(cond, oc)

            for cond, c in sends:
                @pl.when(cond)
                def _(c=c):
                    c.wait_send()

        if _MODE == "comm_only":
            o_ref[...] = q_ref[...]
            return

        for bi in range(b):
            for hi in range(h):
                o_ref[bi, hi] = o_ref[bi, hi] / l_ref[bi, hi]

    Qt = jnp.transpose(Q, (0, 2, 1, 3))
    Kt = jnp.transpose(K, (0, 2, 1, 3))
    Vt = jnp.transpose(V, (0, 2, 1, 3))
    KV = jnp.stack((Kt, Vt), axis=1)

    out = pl.pallas_call(
        body,
        out_shape=jax.ShapeDtypeStruct((b, h, s, d), jnp.float32),
        in_specs=[pl.BlockSpec(memory_space=pltpu.VMEM)] * 2,
        out_specs=pl.BlockSpec(memory_space=pltpu.VMEM),
        scratch_shapes=[
            pltpu.VMEM((NZ, b, 2, h, s, d), jnp.float32),
            pltpu.VMEM((b, h, s, 1), jnp.float32),
            pltpu.SemaphoreType.DMA((NZ - 1,)),
            pltpu.SemaphoreType.DMA((NZ - 1,)),
            pltpu.SemaphoreType.DMA((NZ - 1,)),
            pltpu.SemaphoreType.DMA((NZ - 1,)),
            pltpu.SemaphoreType.DMA((NZ,)),
            pltpu.SemaphoreType.DMA((NZ,)),
            pltpu.SemaphoreType.DMA((NZ,)),
            pltpu.SemaphoreType.DMA((NZ,)),
            pltpu.SemaphoreType.DMA((NZ,)),
            pltpu.SemaphoreType.DMA((NZ,)),
            pltpu.SemaphoreType.DMA((NZ,)),
            pltpu.SemaphoreType.DMA((NZ,)),
            pltpu.SemaphoreType.DMA((NZ,)),
            pltpu.SemaphoreType.DMA((NZ,)),
            pltpu.SemaphoreType.DMA((NZ,)),
            pltpu.SemaphoreType.DMA((NZ,)),
            pltpu.SemaphoreType.DMA((NZ,)),
            pltpu.SemaphoreType.DMA((NZ,)),
        ],
        compiler_params=pltpu.CompilerParams(collective_id=0),
    )(Qt, KV)
    return jnp.transpose(out, (0, 2, 1, 3))


# device time: 75109 ns/iter; 1.0660x vs baseline; 1.0660x over previous
import os

import jax
import jax.numpy as jnp
from jax import lax
from jax.experimental import pallas as pl
from jax.experimental.pallas import tpu as pltpu

NZ = 4
_MODE = os.environ.get("KMODE", "full")
_MESH = pl.DeviceIdType.MESH


def kernel(Q, K, V):
    b, s, h, d = Q.shape
    scale = d ** -0.5
    half = s // 2

    def body(q_ref, k_ref, v_ref, o_ref, kvbuf, l_ref, upsend, dnsend,
             xsd0, xsd1, ysd0, ysd1, yfwd, xfwd,
             zrecv, xrecvd0, xrecvd1, yrecvd0, yrecvd1, yrecvf, xrecvf):
        my_x = lax.axis_index("x")
        my_y = lax.axis_index("y")
        my_z = lax.axis_index("z")
        has_up = my_z < NZ - 1
        has_dn = my_z > 0
        up_dev = (my_x, my_y, jnp.minimum(my_z + 1, NZ - 1))
        dn_dev = (my_x, my_y, jnp.maximum(my_z - 1, 0))
        x_dev = (1 - my_x, my_y, my_z)
        y_dev = (my_x, 1 - my_y, my_z)
        self_dev = (my_x, my_y, my_z)

        bar = pltpu.get_barrier_semaphore()

        @pl.when(has_dn)
        def _():
            pl.semaphore_signal(bar, inc=1, device_id=dn_dev,
                                device_id_type=_MESH)

        @pl.when(has_up)
        def _():
            pl.semaphore_signal(bar, inc=1, device_id=up_dev,
                                device_id_type=_MESH)

        pl.semaphore_signal(bar, inc=1, device_id=x_dev, device_id_type=_MESH)
        pl.semaphore_signal(bar, inc=1, device_id=y_dev, device_id_type=_MESH)
        pl.semaphore_wait(bar, 2)

        @pl.when(has_dn)
        def _():
            pl.semaphore_wait(bar, 1)

        @pl.when(has_up)
        def _():
            pl.semaphore_wait(bar, 1)

        kvbuf[my_z, :, 0] = k_ref[...]
        kvbuf[my_z, :, 1] = v_ref[...]

        sends = []

        def launch(cond, src, dev, ssem, rsem):
            c = pltpu.make_async_remote_copy(
                src_ref=src, dst_ref=src, send_sem=ssem, recv_sem=rsem,
                device_id=dev, device_id_type=_MESH)

            @pl.when(cond)
            def _():
                c.start()

            sends.append((cond, c))

        def wait_recv(cond, dst, rsem):
            c = pltpu.make_async_remote_copy(
                src_ref=dst, dst_ref=dst, send_sem=rsem, recv_sem=rsem,
                device_id=self_dev, device_id_type=_MESH)

            @pl.when(cond)
            def _():
                c.wait_recv()

        def remote_origins():
            for delta in range(1, NZ):
                yield (my_z - delta >= 0, jnp.maximum(my_z - delta, 0))
                yield (my_z + delta <= NZ - 1,
                       jnp.minimum(my_z + delta, NZ - 1))

        def attn_accum(get_k, get_v, init):
            for bi in range(b):
                for hi in range(h):
                    q = q_ref[bi, hi]
                    p = jnp.exp(lax.dot_general(
                        q, get_k(bi, hi), (((1,), (1,)), ((), ())),
                        preferred_element_type=jnp.float32) * scale)
                    lsum = jnp.sum(p, axis=1, keepdims=True)
                    pv = lax.dot_general(
                        p, get_v(bi, hi), (((1,), (0,)), ((), ())),
                        preferred_element_type=jnp.float32)
                    if init:
                        l_ref[bi, hi] = lsum
                        o_ref[bi, hi] = pv
                    else:
                        l_ref[bi, hi] += lsum
                        o_ref[bi, hi] += pv

        def finish_origin(cond, o):
            wait_recv(cond,
                      kvbuf.at[o, 1 - my_x, my_y, :, pl.ds(half, half)],
                      xrecvd1.at[o])
            wait_recv(cond,
                      kvbuf.at[o, my_x, 1 - my_y, :, pl.ds(0, half)],
                      yrecvd0.at[o])
            wait_recv(cond,
                      kvbuf.at[o, 1 - my_x, 1 - my_y, :, pl.ds(0, half)],
                      yrecvf.at[o])
            wait_recv(cond,
                      kvbuf.at[o, 1 - my_x, 1 - my_y, :, pl.ds(half, half)],
                      xrecvf.at[o])

            if _MODE != "comm_only":
                @pl.when(cond)
                def _():
                    attn_accum(lambda bi, hi: kvbuf[o, bi, 0, hi],
                               lambda bi, hi: kvbuf[o, bi, 1, hi],
                               init=False)

        def fwd_process(cond, o):
            wait_recv(cond, kvbuf.at[o, 1 - my_x, my_y, :, pl.ds(0, half)],
                      xrecvd0.at[o])
            launch(cond, kvbuf.at[o, 1 - my_x, my_y, :, pl.ds(0, half)],
                   y_dev, yfwd.at[o], yrecvf.at[o])
            wait_recv(cond,
                      kvbuf.at[o, my_x, 1 - my_y, :, pl.ds(half, half)],
                      yrecvd1.at[o])
            launch(cond,
                   kvbuf.at[o, my_x, 1 - my_y, :, pl.ds(half, half)],
                   x_dev, xfwd.at[o], xrecvf.at[o])

        do_compute = _MODE != "comm_only"

        if _MODE == "compute_only":
            attn_accum(lambda bi, hi: k_ref[bi, hi],
                       lambda bi, hi: v_ref[bi, hi], init=True)
        else:
            pend_fwd = []
            pend_fin = []
            for t in range(NZ - 1):
                uo = my_z - t
                uoc = jnp.maximum(uo, 0)
                do = my_z + t
                doc = jnp.minimum(do, NZ - 1)
                launch(has_up & (uo >= 0), kvbuf.at[uoc, my_x, my_y],
                       up_dev, upsend.at[t], zrecv.at[uoc])
                launch(has_dn & (do <= NZ - 1), kvbuf.at[doc, my_x, my_y],
                       dn_dev, dnsend.at[t], zrecv.at[doc])
                if t == 0 and do_compute:
                    attn_accum(lambda bi, hi: k_ref[bi, hi],
                               lambda bi, hi: v_ref[bi, hi],
                               init=True)
                fin_now = pend_fin
                pend_fin = []
                for cond, oc in pend_fwd:
                    fwd_process(cond, oc)
                    pend_fin.append((cond, oc))
                pend_fwd = []
                for cond, oc in fin_now:
                    finish_origin(cond, oc)
                rb = my_z - 1 - t
                rbc = jnp.maximum(rb, 0)
                ra = my_z + 1 + t
                rac = jnp.minimum(ra, NZ - 1)
                for cond, oc in ((rb >= 0, rbc), (ra <= NZ - 1, rac)):
                    wait_recv(cond, kvbuf.at[oc, my_x, my_y], zrecv.at[oc])
                    launch(cond,
                           kvbuf.at[oc, my_x, my_y, :, pl.ds(0, half)],
                           x_dev, xsd0.at[oc], xrecvd0.at[oc])
                    launch(cond,
                           kvbuf.at[oc, my_x, my_y, :, pl.ds(half, half)],
                           y_dev, ysd1.at[oc], yrecvd1.at[oc])
                    launch(cond,
                           kvbuf.at[oc, my_x, my_y, :, pl.ds(half, half)],
                           x_dev, xsd1.at[oc], xrecvd1.at[oc])
                    launch(cond,
                           kvbuf.at[oc, my_x, my_y, :, pl.ds(0, half)],
                           y_dev, ysd0.at[oc], yrecvd0.at[oc])
                    pend_fwd.append((cond, oc))

            for cond, oc in pend_fwd:
                fwd_process(cond, oc)
                pend_fin.append((cond, oc))
            for cond, oc in pend_fin:
                finish_origin(cond, oc)

            for cond, c in sends:
                @pl.when(cond)
                def _(c=c):
                    c.wait_send()

        if _MODE == "comm_only":
            o_ref[...] = q_ref[...]
            return

        for bi in range(b):
            for hi in range(h):
                o_ref[bi, hi] = o_ref[bi, hi] / l_ref[bi, hi]

    Qt = jnp.transpose(Q, (0, 2, 1, 3))
    Kt = jnp.transpose(K, (0, 2, 1, 3))
    Vt = jnp.transpose(V, (0, 2, 1, 3))

    out = pl.pallas_call(
        body,
        out_shape=jax.ShapeDtypeStruct((b, h, s, d), jnp.float32),
        in_specs=[pl.BlockSpec(memory_space=pltpu.VMEM)] * 3,
        out_specs=pl.BlockSpec(memory_space=pltpu.VMEM),
        scratch_shapes=[
            pltpu.VMEM((NZ, b, 2, h, s, d), jnp.float32),
            pltpu.VMEM((b, h, s, 1), jnp.float32),
            pltpu.SemaphoreType.DMA((NZ - 1,)),
            pltpu.SemaphoreType.DMA((NZ - 1,)),
            pltpu.SemaphoreType.DMA((NZ,)),
            pltpu.SemaphoreType.DMA((NZ,)),
            pltpu.SemaphoreType.DMA((NZ,)),
            pltpu.SemaphoreType.DMA((NZ,)),
            pltpu.SemaphoreType.DMA((NZ,)),
            pltpu.SemaphoreType.DMA((NZ,)),
            pltpu.SemaphoreType.DMA((NZ,)),
            pltpu.SemaphoreType.DMA((NZ,)),
            pltpu.SemaphoreType.DMA((NZ,)),
            pltpu.SemaphoreType.DMA((NZ,)),
            pltpu.SemaphoreType.DMA((NZ,)),
            pltpu.SemaphoreType.DMA((NZ,)),
            pltpu.SemaphoreType.DMA((NZ,)),
        ],
        compiler_params=pltpu.CompilerParams(collective_id=0),
    )(Qt, Kt, Vt)
    return jnp.transpose(out, (0, 2, 1, 3))
